# baseline (speedup 1.0000x reference)
import jax
import jax.numpy as jnp
from jax import lax
from jax.experimental import pallas as pl
from jax.experimental.pallas import tpu as pltpu
from jax.experimental.pallas import tpu_sc as plsc

N_VOCAB = 49408
N_EMBD = 768
N_TOKENS = 77
BATCH = 1024

NC = 2
NS = 16
NW = NC * NS

TOTAL = BATCH * N_TOKENS
PER_W = TOTAL // NW
CHUNK = 16
N_CHUNKS = PER_W // CHUNK
LANES = 16
D_SLICES = N_EMBD // LANES
NBUF = 3
N_BLOCKS = (N_CHUNKS - 1) // NBUF


def _emb_kernel(table_hbm, idx_hbm, pos_hbm, out_hbm,
                idx_v, pos_v, buf0, buf1, buf2,
                gsem0, gsem1, gsem2, wsem0, wsem1, wsem2):
    wid = lax.axis_index("s") * NC + lax.axis_index("c")
    base = wid * PER_W
    bufs = (buf0, buf1, buf2)
    gsems = (gsem0, gsem1, gsem2)
    wsems = (wsem0, wsem1, wsem2)

    pltpu.sync_copy(idx_hbm.at[pl.ds(base, PER_W)], idx_v)
    pltpu.sync_copy(pos_hbm, pos_v)

    def gather_args(c, part):
        src = table_hbm.at[idx_v.at[pl.ds(c * CHUNK, CHUNK)]]
        return src, bufs[part], gsems[part]

    def write_args(c, part):
        dst = out_hbm.at[pl.ds(base + c * CHUNK, CHUNK)]
        return bufs[part], dst, wsems[part]

    def start_gather(c, part):
        src, dst, sem = gather_args(c, part)
        pltpu.async_copy(src, dst, sem)

    def wait_gather(c, part):
        src, dst, sem = gather_args(c, part)
        pltpu.make_async_copy(src, dst, sem).wait()

    def start_write(c, part):
        src, dst, sem = write_args(c, part)
        pltpu.async_copy(src, dst, sem)

    def wait_write(c, part):
        src, dst, sem = write_args(c, part)
        pltpu.make_async_copy(src, dst, sem).wait()

    def add_pos(c, part):
        p0 = lax.rem(c * CHUNK, N_TOKENS)
        buf = bufs[part]

        def row_body(i, _):
            p = p0 + i
            p = p - jnp.where(p >= N_TOKENS, N_TOKENS, 0)
            for j in range(D_SLICES):
                sl = pl.ds(j * LANES, LANES)
                plsc.addupdate(buf.at[i, sl], pos_v[p, sl])
            return 0

        lax.fori_loop(0, CHUNK, row_body, 0, unroll=4)

    def step(c, part, last):
        @pl.when(c >= 2)
        def _():
            wait_write(c - 2, (part + 1) % NBUF)

        if not last:
            start_gather(c + 1, (part + 1) % NBUF)
        wait_gather(c, part)
        add_pos(c, part)
        start_write(c, part)

    start_gather(0, 0)

    def block_body(blk, _):
        for part in range(NBUF):
            step(blk * NBUF + part, part, last=False)
        return 0

    lax.fori_loop(0, N_BLOCKS, block_body, 0, unroll=False)

    step(N_CHUNKS - 1, 0, last=True)
    wait_write(N_CHUNKS - 2, 2)
    wait_write(N_CHUNKS - 1, 0)


@jax.jit
def _emb(x_flat, table, pos):
    mesh = plsc.VectorSubcoreMesh(
        core_axis_name="c", subcore_axis_name="s",
        num_cores=NC, num_subcores=NS,
    )
    f = pl.kernel(
        _emb_kernel,
        out_type=jax.ShapeDtypeStruct((TOTAL, N_EMBD), jnp.float32),
        mesh=mesh,
        scratch_types=[
            pltpu.VMEM((PER_W,), jnp.int32),
            pltpu.VMEM((N_TOKENS, N_EMBD), jnp.float32),
            pltpu.VMEM((CHUNK, N_EMBD), jnp.float32),
            pltpu.VMEM((CHUNK, N_EMBD), jnp.float32),
            pltpu.VMEM((CHUNK, N_EMBD), jnp.float32),
            pltpu.SemaphoreType.DMA,
            pltpu.SemaphoreType.DMA,
            pltpu.SemaphoreType.DMA,
            pltpu.SemaphoreType.DMA,
            pltpu.SemaphoreType.DMA,
            pltpu.SemaphoreType.DMA,
        ],
    )
    return f(table, x_flat, pos)


def kernel(x, token_embedding, position_embedding):
    x_flat = x.reshape(-1).astype(jnp.int32)
    out = _emb(x_flat, token_embedding, position_embedding)
    return out.reshape(BATCH, N_TOKENS, N_EMBD)


# speedup vs baseline: 1.0259x; 1.0259x over previous
import jax
import jax.numpy as jnp
from jax import lax
from jax.experimental import pallas as pl
from jax.experimental.pallas import tpu as pltpu
from jax.experimental.pallas import tpu_sc as plsc

N_VOCAB = 49408
N_EMBD = 768
N_TOKENS = 77
BATCH = 1024

NC = 2
NS = 16
NW = NC * NS

TOTAL = BATCH * N_TOKENS
PER_W = TOTAL // NW
CHUNK = 16
N_CHUNKS = PER_W // CHUNK
LANES = 16
D_SLICES = N_EMBD // LANES
NBUF = 3
N_BLOCKS = (N_CHUNKS - 1) // NBUF


def _emb_kernel(table_hbm, idx_hbm, pos_hbm, out_hbm,
                idx_v, pos_v, buf0, buf1, buf2,
                gsem0, gsem1, gsem2, wsem0, wsem1, wsem2):
    wid = lax.axis_index("s") * NC + lax.axis_index("c")
    base = wid * PER_W
    bufs = (buf0, buf1, buf2)
    gsems = (gsem0, gsem1, gsem2)
    wsems = (wsem0, wsem1, wsem2)

    pltpu.sync_copy(idx_hbm.at[pl.ds(base, PER_W)], idx_v)
    pltpu.sync_copy(pos_hbm, pos_v)

    def gather_args(c, part):
        src = table_hbm.at[idx_v.at[pl.ds(c * CHUNK, CHUNK)]]
        return src, bufs[part], gsems[part]

    def write_args(c, part):
        dst = out_hbm.at[pl.ds(base + c * CHUNK, CHUNK)]
        return bufs[part], dst, wsems[part]

    def start_gather(c, part):
        src, dst, sem = gather_args(c, part)
        pltpu.async_copy(src, dst, sem)

    def wait_gather(c, part):
        src, dst, sem = gather_args(c, part)
        pltpu.make_async_copy(src, dst, sem).wait()

    def start_write(c, part):
        src, dst, sem = write_args(c, part)
        pltpu.async_copy(src, dst, sem)

    def wait_write(c, part):
        src, dst, sem = write_args(c, part)
        pltpu.make_async_copy(src, dst, sem).wait()

    def add_pos(c, part):
        p0 = lax.rem(c * CHUNK, N_TOKENS)
        buf = bufs[part]

        def row_body(i, _):
            p = p0 + i
            p = p - jnp.where(p >= N_TOKENS, N_TOKENS, 0)
            for j in range(D_SLICES):
                sl = pl.ds(j * LANES, LANES)
                plsc.addupdate(buf.at[i, sl], pos_v[p, sl])
            return 0

        lax.fori_loop(0, CHUNK, row_body, 0, unroll=False)

    def step(c, part, last):
        @pl.when(c >= 2)
        def _():
            wait_write(c - 2, (part + 1) % NBUF)

        if not last:
            start_gather(c + 1, (part + 1) % NBUF)
        wait_gather(c, part)
        add_pos(c, part)
        start_write(c, part)

    start_gather(0, 0)

    def block_body(blk, _):
        for part in range(NBUF):
            step(blk * NBUF + part, part, last=False)
        return 0

    lax.fori_loop(0, N_BLOCKS, block_body, 0, unroll=False)

    step(N_CHUNKS - 1, 0, last=True)
    wait_write(N_CHUNKS - 2, 2)
    wait_write(N_CHUNKS - 1, 0)


@jax.jit
def _emb(x_flat, table, pos):
    mesh = plsc.VectorSubcoreMesh(
        core_axis_name="c", subcore_axis_name="s",
        num_cores=NC, num_subcores=NS,
    )
    f = pl.kernel(
        _emb_kernel,
        out_type=jax.ShapeDtypeStruct((TOTAL, N_EMBD), jnp.float32),
        mesh=mesh,
        scratch_types=[
            pltpu.VMEM((PER_W,), jnp.int32),
            pltpu.VMEM((N_TOKENS, N_EMBD), jnp.float32),
            pltpu.VMEM((CHUNK, N_EMBD), jnp.float32),
            pltpu.VMEM((CHUNK, N_EMBD), jnp.float32),
            pltpu.VMEM((CHUNK, N_EMBD), jnp.float32),
            pltpu.SemaphoreType.DMA,
            pltpu.SemaphoreType.DMA,
            pltpu.SemaphoreType.DMA,
            pltpu.SemaphoreType.DMA,
            pltpu.SemaphoreType.DMA,
            pltpu.SemaphoreType.DMA,
        ],
    )
    return f(table, x_flat, pos)


def kernel(x, token_embedding, position_embedding):
    x_flat = x.reshape(-1).astype(jnp.int32)
    out = _emb(x_flat, token_embedding, position_embedding)
    return out.reshape(BATCH, N_TOKENS, N_EMBD)


# speedup vs baseline: 1.2908x; 1.2582x over previous
import jax
import jax.numpy as jnp
from jax import lax
from jax.experimental import pallas as pl
from jax.experimental.pallas import tpu as pltpu
from jax.experimental.pallas import tpu_sc as plsc

N_VOCAB = 49408
N_EMBD = 768
N_TOKENS = 77
BATCH = 1024

NC = 2
NS = 16
NW = NC * NS

ROWS_PER_W = BATCH // NW
T_PAD = 80
PARTS = ((0, 24), (24, 24), (48, 32))
LANES = 16
D_SLICES = N_EMBD // LANES


def _emb_kernel(table_hbm, idx_hbm, pos_hbm, out_hbm,
                idx_v, pos_v, buf0, buf1, buf2,
                gsem0, gsem1, gsem2, wsem0, wsem1, wsem2):
    wid = lax.axis_index("s") * NC + lax.axis_index("c")
    row0 = wid * ROWS_PER_W
    bufs = (buf0, buf1, buf2)
    gsems = (gsem0, gsem1, gsem2)
    wsems = (wsem0, wsem1, wsem2)

    pltpu.sync_copy(idx_hbm.at[pl.ds(wid * ROWS_PER_W * T_PAD,
                                     ROWS_PER_W * T_PAD)], idx_v)
    pltpu.sync_copy(pos_hbm, pos_v)

    def gather_args(k, part):
        off, ln = PARTS[part]
        src = table_hbm.at[idx_v.at[pl.ds(k * T_PAD + off, ln)]]
        return src, bufs[part], gsems[part]

    def write_args(k, part):
        off, ln = PARTS[part]
        dst = out_hbm.at[pl.ds((row0 + k) * T_PAD + off, ln)]
        return bufs[part], dst, wsems[part]

    def start_gather(k, part):
        src, dst, sem = gather_args(k, part)
        pltpu.async_copy(src, dst, sem)

    def wait_gather(k, part):
        src, dst, sem = gather_args(k, part)
        pltpu.make_async_copy(src, dst, sem).wait()

    def start_write(k, part):
        src, dst, sem = write_args(k, part)
        pltpu.async_copy(src, dst, sem)

    def wait_write(k, part):
        src, dst, sem = write_args(k, part)
        pltpu.make_async_copy(src, dst, sem).wait()

    def add_pos(part):
        off, ln = PARTS[part]
        buf = bufs[part]

        def row_body(i, _):
            for j in range(D_SLICES):
                sl = pl.ds(j * LANES, LANES)
                plsc.addupdate(buf.at[i, sl], pos_v[off + i, sl])
            return 0

        lax.fori_loop(0, ln, row_body, 0, unroll=False)

    start_gather(0, 0)

    def row_loop(k, _):
        for part in range(3):
            if part == 2:
                wait_write(k, 0)
            else:

                @pl.when(k >= 1)
                def _():
                    wait_write(k - 1, part + 1)

            if part == 2:

                @pl.when(k + 1 < ROWS_PER_W)
                def _():
                    start_gather(k + 1, 0)
            else:
                start_gather(k, part + 1)

            wait_gather(k, part)
            add_pos(part)
            start_write(k, part)
        return 0

    lax.fori_loop(0, ROWS_PER_W, row_loop, 0, unroll=False)

    wait_write(ROWS_PER_W - 1, 1)
    wait_write(ROWS_PER_W - 1, 2)


@jax.jit
def _emb(x_pad_flat, table, pos):
    mesh = plsc.VectorSubcoreMesh(
        core_axis_name="c", subcore_axis_name="s",
        num_cores=NC, num_subcores=NS,
    )
    f = pl.kernel(
        _emb_kernel,
        out_type=jax.ShapeDtypeStruct((BATCH * T_PAD, N_EMBD),
                                      jnp.float32),
        mesh=mesh,
        scratch_types=[
            pltpu.VMEM((ROWS_PER_W * T_PAD,), jnp.int32),
            pltpu.VMEM((T_PAD, N_EMBD), jnp.float32),
            pltpu.VMEM((PARTS[0][1], N_EMBD), jnp.float32),
            pltpu.VMEM((PARTS[1][1], N_EMBD), jnp.float32),
            pltpu.VMEM((PARTS[2][1], N_EMBD), jnp.float32),
            pltpu.SemaphoreType.DMA,
            pltpu.SemaphoreType.DMA,
            pltpu.SemaphoreType.DMA,
            pltpu.SemaphoreType.DMA,
            pltpu.SemaphoreType.DMA,
            pltpu.SemaphoreType.DMA,
        ],
    )
    return f(table, x_pad_flat, pos)


def kernel(x, token_embedding, position_embedding):
    x_pad = jnp.pad(x.astype(jnp.int32), ((0, 0), (0, T_PAD - N_TOKENS)))
    pos_pad = jnp.pad(position_embedding, ((0, T_PAD - N_TOKENS), (0, 0)))
    out = _emb(x_pad.reshape(-1), token_embedding, pos_pad)
    return out.reshape(BATCH, T_PAD, N_EMBD)[:, :N_TOKENS, :]


# speedup vs baseline: 1.3303x; 1.0306x over previous
import jax
import jax.numpy as jnp
from jax import lax
from jax.experimental import pallas as pl
from jax.experimental.pallas import tpu as pltpu
from jax.experimental.pallas import tpu_sc as plsc

N_VOCAB = 49408
N_EMBD = 768
N_TOKENS = 77
BATCH = 1024

NC = 2
NS = 16
NW = NC * NS

ROWS_PER_W = BATCH // NW
T_PAD = 80
PARTS = ((0, 24), (24, 24), (48, 32))
LANES = 16
D_SLICES = N_EMBD // LANES


def _emb_kernel(table_hbm, idx_hbm, pos_hbm, out_hbm,
                idx_v, pos_v, buf0, buf1, buf2,
                gsem0, gsem1, gsem2, wsem0, wsem1, wsem2):
    wid = lax.axis_index("s") * NC + lax.axis_index("c")
    row0 = wid * ROWS_PER_W
    bufs = (buf0, buf1, buf2)
    gsems = (gsem0, gsem1, gsem2)
    wsems = (wsem0, wsem1, wsem2)

    pltpu.sync_copy(idx_hbm.at[pl.ds(wid * ROWS_PER_W * T_PAD,
                                     ROWS_PER_W * T_PAD)], idx_v)
    pltpu.sync_copy(pos_hbm, pos_v)

    def gather_args(k, part):
        off, ln = PARTS[part]
        src = table_hbm.at[idx_v.at[pl.ds(k * T_PAD + off, ln)]]
        return src, bufs[part], gsems[part]

    def write_args(k, part):
        off, ln = PARTS[part]
        dst = out_hbm.at[pl.ds((row0 + k) * T_PAD + off, ln)]
        return bufs[part], dst, wsems[part]

    def start_gather(k, part):
        src, dst, sem = gather_args(k, part)
        pltpu.async_copy(src, dst, sem)

    def wait_gather(k, part):
        src, dst, sem = gather_args(k, part)
        pltpu.make_async_copy(src, dst, sem).wait()

    def start_write(k, part):
        src, dst, sem = write_args(k, part)
        pltpu.async_copy(src, dst, sem)

    def wait_write(k, part):
        src, dst, sem = write_args(k, part)
        pltpu.make_async_copy(src, dst, sem).wait()

    def add_pos(part):
        off, ln = PARTS[part]
        buf = bufs[part]

        def row_body(i, _):
            for j0 in range(0, D_SLICES, 8):
                vals = [pos_v[off + i, pl.ds((j0 + j) * LANES, LANES)]
                        for j in range(8)]
                for j in range(8):
                    plsc.addupdate(
                        buf.at[i, pl.ds((j0 + j) * LANES, LANES)], vals[j])
            return 0

        lax.fori_loop(0, ln, row_body, 0, unroll=False)

    start_gather(0, 0)

    def row_loop(k, _):
        for part in range(3):
            if part == 2:
                wait_write(k, 0)
            else:

                @pl.when(k >= 1)
                def _():
                    wait_write(k - 1, part + 1)

            if part == 2:

                @pl.when(k + 1 < ROWS_PER_W)
                def _():
                    start_gather(k + 1, 0)
            else:
                start_gather(k, part + 1)

            wait_gather(k, part)
            add_pos(part)
            start_write(k, part)
        return 0

    lax.fori_loop(0, ROWS_PER_W, row_loop, 0, unroll=False)

    wait_write(ROWS_PER_W - 1, 1)
    wait_write(ROWS_PER_W - 1, 2)


@jax.jit
def _emb(x_pad_flat, table, pos):
    mesh = plsc.VectorSubcoreMesh(
        core_axis_name="c", subcore_axis_name="s",
        num_cores=NC, num_subcores=NS,
    )
    f = pl.kernel(
        _emb_kernel,
        out_type=jax.ShapeDtypeStruct((BATCH * T_PAD, N_EMBD),
                                      jnp.float32),
        mesh=mesh,
        scratch_types=[
            pltpu.VMEM((ROWS_PER_W * T_PAD,), jnp.int32),
            pltpu.VMEM((T_PAD, N_EMBD), jnp.float32),
            pltpu.VMEM((PARTS[0][1], N_EMBD), jnp.float32),
            pltpu.VMEM((PARTS[1][1], N_EMBD), jnp.float32),
            pltpu.VMEM((PARTS[2][1], N_EMBD), jnp.float32),
            pltpu.SemaphoreType.DMA,
            pltpu.SemaphoreType.DMA,
            pltpu.SemaphoreType.DMA,
            pltpu.SemaphoreType.DMA,
            pltpu.SemaphoreType.DMA,
            pltpu.SemaphoreType.DMA,
        ],
    )
    return f(table, x_pad_flat, pos)


def kernel(x, token_embedding, position_embedding):
    x_pad = jnp.pad(x.astype(jnp.int32), ((0, 0), (0, T_PAD - N_TOKENS)))
    pos_pad = jnp.pad(position_embedding, ((0, T_PAD - N_TOKENS), (0, 0)))
    out = _emb(x_pad.reshape(-1), token_embedding, pos_pad)
    return out.reshape(BATCH, T_PAD, N_EMBD)[:, :N_TOKENS, :]


# speedup vs baseline: 1.3306x; 1.0003x over previous
import jax
import jax.numpy as jnp
from jax import lax
from jax.experimental import pallas as pl
from jax.experimental.pallas import tpu as pltpu
from jax.experimental.pallas import tpu_sc as plsc

N_VOCAB = 49408
N_EMBD = 768
N_TOKENS = 77
BATCH = 1024

NC = 2
NS = 16
NW = NC * NS

ROWS_PER_W = BATCH // NW
T_PAD = 80
PARTS = ((0, 24), (24, 24), (48, 32))
LANES = 16
D_SLICES = N_EMBD // LANES


def _emb_kernel(table_hbm, idx_hbm, pos_hbm, out_hbm,
                idx_v, pos_v, buf0, buf1, buf2,
                gsem0, gsem1, gsem2, wsem0, wsem1, wsem2):
    wid = lax.axis_index("s") * NC + lax.axis_index("c")
    row0 = wid * ROWS_PER_W
    bufs = (buf0, buf1, buf2)
    gsems = (gsem0, gsem1, gsem2)
    wsems = (wsem0, wsem1, wsem2)

    pltpu.sync_copy(idx_hbm.at[pl.ds(wid * ROWS_PER_W * T_PAD,
                                     ROWS_PER_W * T_PAD)], idx_v)
    pltpu.sync_copy(pos_hbm, pos_v)

    def gather_args(k, part):
        off, ln = PARTS[part]
        src = table_hbm.at[idx_v.at[pl.ds(k * T_PAD + off, ln)]]
        return src, bufs[part], gsems[part]

    def write_args(k, part):
        off, ln = PARTS[part]
        dst = out_hbm.at[pl.ds((row0 + k) * T_PAD + off, ln)]
        return bufs[part], dst, wsems[part]

    def start_gather(k, part):
        src, dst, sem = gather_args(k, part)
        pltpu.async_copy(src, dst, sem)

    def wait_gather(k, part):
        src, dst, sem = gather_args(k, part)
        pltpu.make_async_copy(src, dst, sem).wait()

    def start_write(k, part):
        src, dst, sem = write_args(k, part)
        pltpu.async_copy(src, dst, sem)

    def wait_write(k, part):
        src, dst, sem = write_args(k, part)
        pltpu.make_async_copy(src, dst, sem).wait()

    def add_pos(part):
        off, ln = PARTS[part]
        buf = bufs[part]

        def row_body(i, _):
            for j0 in range(0, D_SLICES, 16):
                vals = [pos_v[off + i, pl.ds((j0 + j) * LANES, LANES)]
                        for j in range(16)]
                for j in range(16):
                    plsc.addupdate(
                        buf.at[i, pl.ds((j0 + j) * LANES, LANES)], vals[j])
            return 0

        lax.fori_loop(0, ln, row_body, 0, unroll=False)

    start_gather(0, 0)

    def row_loop(k, _):
        for part in range(3):
            if part == 2:
                wait_write(k, 0)
            else:

                @pl.when(k >= 1)
                def _():
                    wait_write(k - 1, part + 1)

            if part == 2:

                @pl.when(k + 1 < ROWS_PER_W)
                def _():
                    start_gather(k + 1, 0)
            else:
                start_gather(k, part + 1)

            wait_gather(k, part)
            add_pos(part)
            start_write(k, part)
        return 0

    lax.fori_loop(0, ROWS_PER_W, row_loop, 0, unroll=False)

    wait_write(ROWS_PER_W - 1, 1)
    wait_write(ROWS_PER_W - 1, 2)


@jax.jit
def _emb(x_pad_flat, table, pos):
    mesh = plsc.VectorSubcoreMesh(
        core_axis_name="c", subcore_axis_name="s",
        num_cores=NC, num_subcores=NS,
    )
    f = pl.kernel(
        _emb_kernel,
        out_type=jax.ShapeDtypeStruct((BATCH * T_PAD, N_EMBD),
                                      jnp.float32),
        mesh=mesh,
        scratch_types=[
            pltpu.VMEM((ROWS_PER_W * T_PAD,), jnp.int32),
            pltpu.VMEM((T_PAD, N_EMBD), jnp.float32),
            pltpu.VMEM((PARTS[0][1], N_EMBD), jnp.float32),
            pltpu.VMEM((PARTS[1][1], N_EMBD), jnp.float32),
            pltpu.VMEM((PARTS[2][1], N_EMBD), jnp.float32),
            pltpu.SemaphoreType.DMA,
            pltpu.SemaphoreType.DMA,
            pltpu.SemaphoreType.DMA,
            pltpu.SemaphoreType.DMA,
            pltpu.SemaphoreType.DMA,
            pltpu.SemaphoreType.DMA,
        ],
    )
    return f(table, x_pad_flat, pos)


def kernel(x, token_embedding, position_embedding):
    x_pad = jnp.pad(x.astype(jnp.int32), ((0, 0), (0, T_PAD - N_TOKENS)))
    pos_pad = jnp.pad(position_embedding, ((0, T_PAD - N_TOKENS), (0, 0)))
    out = _emb(x_pad.reshape(-1), token_embedding, pos_pad)
    return out.reshape(BATCH, T_PAD, N_EMBD)[:, :N_TOKENS, :]
